# Initial kernel scaffold; baseline (speedup 1.0000x reference)
import functools

import jax
import jax.numpy as jnp
from jax import lax
from jax.experimental import pallas as pl
from jax.experimental.pallas import tpu as pltpu
from jax.experimental.pallas import tpu_sc as plsc

N_NODES = 10000
N_EDGES = 320000
D_FEAT = 128
D_EDGE = 16
D_OUT = 128

NC = 2
NS = 16
NW = NC * NS
E_PER_W = N_EDGES // NW
CHUNK = 80
N_CHUNKS = E_PER_W // CHUNK
ROWS_PER_TILE = N_NODES // NS
ZROWS = 125


def _xm_body(x_ref, w_ref, b_ref, o_ref):
    o_ref[...] = (
        jnp.dot(x_ref[...], w_ref[...], preferred_element_type=jnp.float32)
        + b_ref[...]
    )


def _em_body(ea_ref, w_ref, o_ref):
    o_ref[...] = jnp.dot(ea_ref[...], w_ref[...], preferred_element_type=jnp.float32)


def _upd_body(x_ref, a_ref, w1_ref, w2_ref, b_ref, o_ref):
    agg = a_ref[0] + a_ref[1]
    t = (
        jnp.dot(x_ref[...], w1_ref[...], preferred_element_type=jnp.float32)
        + jnp.dot(agg, w2_ref[...], preferred_element_type=jnp.float32)
        + b_ref[...]
    )
    o_ref[...] = jnp.maximum(t, 0.0)


def _sc_body(xm_hbm, em_hbm, src_hbm, dst_hbm, out_hbm,
             idx_s, idx_d, rows, em_buf, zbuf, agg_sh, sem_g, sem_e):
    c = lax.axis_index("c")
    s = lax.axis_index("s")
    w = s * NC + c

    zero16 = jnp.zeros((16,), jnp.float32)

    def zrow(i, carry):
        for j in range(8):
            zbuf[i, pl.ds(j * 16, 16)] = zero16
        return carry

    lax.fori_loop(0, ZROWS, zrow, 0, unroll=False)
    for r in range(ROWS_PER_TILE // ZROWS):
        pltpu.sync_copy(zbuf, agg_sh.at[pl.ds(s * ROWS_PER_TILE + r * ZROWS, ZROWS)])
    plsc.subcore_barrier()

    def chunk(i, carry):
        base = w * E_PER_W + i * CHUNK
        pltpu.sync_copy(src_hbm.at[pl.ds(base, CHUNK)], idx_s)
        pltpu.sync_copy(dst_hbm.at[pl.ds(base, CHUNK)], idx_d)
        g = pltpu.async_copy(xm_hbm.at[idx_s], rows, sem_g)
        e = pltpu.async_copy(em_hbm.at[pl.ds(base, CHUNK)], em_buf, sem_e)
        e.wait()
        g.wait()

        def erow(r, carry2):
            for j in range(8):
                sl = pl.ds(j * 16, 16)
                v = rows[r, sl] + em_buf[r, sl]
                rows[r, sl] = jnp.maximum(v, 0.0)
            return carry2

        lax.fori_loop(0, CHUNK, erow, 0, unroll=False)
        pltpu.sync_copy(rows, agg_sh.at[idx_d], add=True)
        return carry

    lax.fori_loop(0, N_CHUNKS, chunk, 0, unroll=False)

    plsc.subcore_barrier()
    off = s * ROWS_PER_TILE
    pltpu.sync_copy(agg_sh.at[pl.ds(off, ROWS_PER_TILE)],
                    out_hbm.at[c, pl.ds(off, ROWS_PER_TILE)])


@jax.jit
def _run(x, src, dst, edge_attr, W_msg, b_msg, W_upd, b_upd):
    W1 = W_msg[:D_FEAT]
    W2 = W_msg[D_FEAT:]
    Wu1 = W_upd[:D_FEAT]
    Wu2 = W_upd[D_FEAT:]
    b_msg2 = b_msg.reshape(1, D_OUT)
    b_upd2 = b_upd.reshape(1, D_OUT)

    xm = pl.pallas_call(
        _xm_body,
        out_shape=jax.ShapeDtypeStruct((N_NODES, D_OUT), jnp.float32),
    )(x, W1, b_msg2)

    EB = 6400
    em = pl.pallas_call(
        _em_body,
        grid=(N_EDGES // EB,),
        in_specs=[
            pl.BlockSpec((EB, D_EDGE), lambda i: (i, 0)),
            pl.BlockSpec((D_EDGE, D_OUT), lambda i: (0, 0)),
        ],
        out_specs=pl.BlockSpec((EB, D_OUT), lambda i: (i, 0)),
        out_shape=jax.ShapeDtypeStruct((N_EDGES, D_OUT), jnp.float32),
    )(edge_attr, W2)

    mesh = plsc.VectorSubcoreMesh(
        core_axis_name="c", subcore_axis_name="s", num_cores=NC, num_subcores=NS
    )
    agg2 = pl.kernel(
        _sc_body,
        out_type=jax.ShapeDtypeStruct((NC, N_NODES, D_OUT), jnp.float32),
        mesh=mesh,
        scratch_types=[
            pltpu.VMEM((CHUNK,), jnp.int32),
            pltpu.VMEM((CHUNK,), jnp.int32),
            pltpu.VMEM((CHUNK, D_OUT), jnp.float32),
            pltpu.VMEM((CHUNK, D_OUT), jnp.float32),
            pltpu.VMEM((ZROWS, D_OUT), jnp.float32),
            pltpu.VMEM_SHARED((N_NODES, D_OUT), jnp.float32),
            pltpu.SemaphoreType.DMA,
            pltpu.SemaphoreType.DMA,
        ],
    )(xm, em, src, dst)

    NB = 2000
    out = pl.pallas_call(
        _upd_body,
        grid=(N_NODES // NB,),
        in_specs=[
            pl.BlockSpec((NB, D_FEAT), lambda i: (i, 0)),
            pl.BlockSpec((NC, NB, D_OUT), lambda i: (0, i, 0)),
            pl.BlockSpec((D_FEAT, D_OUT), lambda i: (0, 0)),
            pl.BlockSpec((D_OUT, D_OUT), lambda i: (0, 0)),
            pl.BlockSpec((1, D_OUT), lambda i: (0, 0)),
        ],
        out_specs=pl.BlockSpec((NB, D_OUT), lambda i: (i, 0)),
        out_shape=jax.ShapeDtypeStruct((N_NODES, D_OUT), jnp.float32),
    )(x, agg2, Wu1, Wu2, b_upd2)
    return out


def kernel(x, edge_index, edge_attr, W_msg, b_msg, W_upd, b_upd):
    src = edge_index[0].astype(jnp.int32)
    dst = edge_index[1].astype(jnp.int32)
    return _run(x, src, dst, edge_attr, W_msg, b_msg, W_upd, b_upd)


# speedup vs baseline: 3.0836x; 3.0836x over previous
import functools

import jax
import jax.numpy as jnp
from jax import lax
from jax.experimental import pallas as pl
from jax.experimental.pallas import tpu as pltpu
from jax.experimental.pallas import tpu_sc as plsc

N_NODES = 10000
N_EDGES = 320000
D_FEAT = 128
D_EDGE = 16
D_OUT = 128

NC = 2
NS = 16
NW = NC * NS
E_PER_W = N_EDGES // NW
CHUNK = 80
N_CHUNKS = E_PER_W // CHUNK
N_PAD = 10240
ROWS_PER_TILE = N_PAD // NS
ZROWS = 128


def _xm_body(x_ref, w_ref, b_ref, o_ref):
    o_ref[...] = (
        jnp.dot(x_ref[...], w_ref[...], preferred_element_type=jnp.float32)
        + b_ref[...]
    )


def _em_body(ea_ref, w_ref, o_ref):
    o_ref[...] = jnp.dot(ea_ref[...], w_ref[...], preferred_element_type=jnp.float32)


def _upd_body(x_ref, a_ref, w1_ref, w2_ref, b_ref, o_ref):
    agg = a_ref[0] + a_ref[1]
    t = (
        jnp.dot(x_ref[...], w1_ref[...], preferred_element_type=jnp.float32)
        + jnp.dot(agg, w2_ref[...], preferred_element_type=jnp.float32)
        + b_ref[...]
    )
    o_ref[...] = jnp.maximum(t, 0.0)


def _sc_body(xm_hbm, em_hbm, src_hbm, dst_hbm, out_hbm,
             idx_s, idx_d, rows, em_buf, zbuf, agg_sh, sem_g, sem_e):
    c = lax.axis_index("c")
    s = lax.axis_index("s")
    w = s * NC + c

    zero16 = jnp.zeros((16,), jnp.float32)

    def zrow(i, carry):
        for j in range(8):
            zbuf[i, pl.ds(j * 16, 16)] = zero16
        return carry

    lax.fori_loop(0, ZROWS, zrow, 0, unroll=False)
    for r in range(ROWS_PER_TILE // ZROWS):
        pltpu.sync_copy(zbuf, agg_sh.at[pl.ds(s * ROWS_PER_TILE + r * ZROWS, ZROWS)])
    plsc.subcore_barrier()

    def chunk(i, carry):
        base = w * E_PER_W + i * CHUNK
        pltpu.sync_copy(src_hbm.at[pl.ds(base, CHUNK)], idx_s)
        pltpu.sync_copy(dst_hbm.at[pl.ds(base, CHUNK)], idx_d)
        g = pltpu.async_copy(xm_hbm.at[idx_s], rows, sem_g)
        e = pltpu.async_copy(em_hbm.at[pl.ds(base, CHUNK)], em_buf, sem_e)
        e.wait()
        g.wait()

        def erow(r, carry2):
            for j in range(8):
                sl = pl.ds(j * 16, 16)
                v = rows[r, sl] + em_buf[r, sl]
                rows[r, sl] = jnp.maximum(v, 0.0)
            return carry2

        lax.fori_loop(0, CHUNK, erow, 0, unroll=False)
        pltpu.sync_copy(rows, agg_sh.at[idx_d], add=True)
        return carry

    lax.fori_loop(0, N_CHUNKS, chunk, 0, unroll=False)

    plsc.subcore_barrier()
    off = s * ROWS_PER_TILE
    pltpu.sync_copy(agg_sh.at[pl.ds(off, ROWS_PER_TILE)],
                    out_hbm.at[c, pl.ds(off, ROWS_PER_TILE)])


@jax.jit
def _run(x, src, dst, edge_attr, W_msg, b_msg, W_upd, b_upd):
    W1 = W_msg[:D_FEAT]
    W2 = W_msg[D_FEAT:]
    Wu1 = W_upd[:D_FEAT]
    Wu2 = W_upd[D_FEAT:]
    b_msg2 = b_msg.reshape(1, D_OUT)
    b_upd2 = b_upd.reshape(1, D_OUT)

    xm = pl.pallas_call(
        _xm_body,
        out_shape=jax.ShapeDtypeStruct((N_NODES, D_OUT), jnp.float32),
    )(x, W1, b_msg2)

    EB = 6400
    em = pl.pallas_call(
        _em_body,
        grid=(N_EDGES // EB,),
        in_specs=[
            pl.BlockSpec((EB, D_EDGE), lambda i: (i, 0)),
            pl.BlockSpec((D_EDGE, D_OUT), lambda i: (0, 0)),
        ],
        out_specs=pl.BlockSpec((EB, D_OUT), lambda i: (i, 0)),
        out_shape=jax.ShapeDtypeStruct((N_EDGES, D_OUT), jnp.float32),
    )(edge_attr, W2)

    mesh = plsc.VectorSubcoreMesh(
        core_axis_name="c", subcore_axis_name="s", num_cores=NC, num_subcores=NS
    )
    agg2 = pl.kernel(
        _sc_body,
        out_type=jax.ShapeDtypeStruct((NC, N_PAD, D_OUT), jnp.float32),
        mesh=mesh,
        scratch_types=[
            pltpu.VMEM((CHUNK,), jnp.int32),
            pltpu.VMEM((CHUNK,), jnp.int32),
            pltpu.VMEM((CHUNK, D_OUT), jnp.float32),
            pltpu.VMEM((CHUNK, D_OUT), jnp.float32),
            pltpu.VMEM((ZROWS, D_OUT), jnp.float32),
            pltpu.VMEM_SHARED((N_PAD, D_OUT), jnp.float32),
            pltpu.SemaphoreType.DMA,
            pltpu.SemaphoreType.DMA,
        ],
    )(xm, em, src, dst)
    agg2 = agg2[:, :N_NODES]

    NB = 2000
    out = pl.pallas_call(
        _upd_body,
        grid=(N_NODES // NB,),
        in_specs=[
            pl.BlockSpec((NB, D_FEAT), lambda i: (i, 0)),
            pl.BlockSpec((NC, NB, D_OUT), lambda i: (0, i, 0)),
            pl.BlockSpec((D_FEAT, D_OUT), lambda i: (0, 0)),
            pl.BlockSpec((D_OUT, D_OUT), lambda i: (0, 0)),
            pl.BlockSpec((1, D_OUT), lambda i: (0, 0)),
        ],
        out_specs=pl.BlockSpec((NB, D_OUT), lambda i: (i, 0)),
        out_shape=jax.ShapeDtypeStruct((N_NODES, D_OUT), jnp.float32),
    )(x, agg2, Wu1, Wu2, b_upd2)
    return out


def kernel(x, edge_index, edge_attr, W_msg, b_msg, W_upd, b_upd):
    src = edge_index[0].astype(jnp.int32)
    dst = edge_index[1].astype(jnp.int32)
    return _run(x, src, dst, edge_attr, W_msg, b_msg, W_upd, b_upd)
